# Initial kernel scaffold; baseline (speedup 1.0000x reference)
import jax
import jax.numpy as jnp
from jax.experimental import pallas as pl


def kernel(x, child_idx, W_top, W_left, W_right):
    raise NotImplementedError("write your pallas kernel here")


# speedup vs baseline: 1.8777x; 1.8777x over previous
import functools

import jax
import jax.numpy as jnp
from jax import lax
from jax.experimental import pallas as pl
from jax.experimental.pallas import tpu as pltpu
from jax.experimental.pallas import tpu_sc as plsc

NW = 32
CN = 4
LANES = 16


def _sc_gather_reduce(x, idx_flat, npw, nch):
  n_pad = NW * npw
  f = x.shape[1]
  k = idx_flat.shape[0] // n_pad
  ci = CN * k
  inv = 1.0 / (k - 1)
  mesh = plsc.VectorSubcoreMesh(core_axis_name="c", subcore_axis_name="s")

  @functools.partial(
      pl.kernel,
      out_type=[jax.ShapeDtypeStruct((n_pad, f), jnp.float32),
                jax.ShapeDtypeStruct((n_pad, f), jnp.float32)],
      mesh=mesh,
      scratch_types=[
          pltpu.VMEM((npw * k,), jnp.int32),
          pltpu.VMEM((2, ci, f), jnp.float32),
          pltpu.VMEM((npw, f), jnp.float32),
          pltpu.VMEM((npw, f), jnp.float32),
          pltpu.SemaphoreType.DMA,
          pltpu.SemaphoreType.DMA,
      ],
  )
  def sc_kernel(x_hbm, idx_hbm, gl_hbm, gr_hbm, idxv, rows, glv, grv,
                sem0, sem1):
    sems = (sem0, sem1)
    wid = lax.axis_index("s") * 2 + lax.axis_index("c")
    base = wid * npw

    pltpu.sync_copy(idx_hbm.at[pl.ds(base * k, npw * k)], idxv)

    def gather(c, b):
      return pltpu.make_async_copy(
          x_hbm.at[idxv.at[pl.ds(c * ci, ci)]], rows.at[b], sems[b])

    gather(0, 0).start()
    gather(1, 1).start()

    def do_chunk(c, b):
      gather(c, b).wait()

      @pl.loop(0, CN)
      def _(i):
        node = c * CN + i
        r0 = i * k
        for fo in range(f // LANES):
          sl = pl.ds(fo * LANES, LANES)
          v = rows[b, r0 + (k - 1), sl]
          s = v
          w = v
          for j in range(k - 2, 0, -1):
            v = rows[b, r0 + j, sl]
            s = s + v
            w = w + s
          s = s + rows[b, r0, sl]
          gr = w * inv
          glv[node, sl] = s - gr
          grv[node, sl] = gr

      nxt = c + 2

      @pl.when(nxt < nch)
      def _():
        gather(nxt, b).start()

    @pl.loop(0, nch, step=2)
    def _(c):
      do_chunk(c, 0)
      do_chunk(c + 1, 1)

    pltpu.sync_copy(glv, gl_hbm.at[pl.ds(base, npw)])
    pltpu.sync_copy(grv, gr_hbm.at[pl.ds(base, npw)])

  return sc_kernel(x, idx_flat)


def _tc_combine(xp, gl, gr, wt_t, wl_t, wr_t, bm):
  n_pad, f = xp.shape

  def body(x_ref, gl_ref, gr_ref, wt_ref, wl_ref, wr_ref, o_ref):
    acc = jnp.dot(x_ref[...], wt_ref[...], preferred_element_type=jnp.float32)
    acc += jnp.dot(gl_ref[...], wl_ref[...], preferred_element_type=jnp.float32)
    acc += jnp.dot(gr_ref[...], wr_ref[...], preferred_element_type=jnp.float32)
    o_ref[...] = acc

  row_spec = pl.BlockSpec((bm, f), lambda i: (i, 0))
  w_spec = pl.BlockSpec((f, f), lambda i: (0, 0))
  return pl.pallas_call(
      body,
      grid=(n_pad // bm,),
      in_specs=[row_spec, row_spec, row_spec, w_spec, w_spec, w_spec],
      out_specs=row_spec,
      out_shape=jax.ShapeDtypeStruct((n_pad, f), jnp.float32),
  )(xp, gl, gr, wt_t, wl_t, wr_t)


def kernel(x, child_idx, W_top, W_left, W_right):
  n, _ = x.shape
  nch = -(-n // (NW * CN))
  nch += nch % 2
  npw = nch * CN
  n_pad = NW * npw
  idx = jnp.pad(child_idx.astype(jnp.int32), ((0, n_pad - n), (0, 0)))
  gl, gr = _sc_gather_reduce(x, idx.reshape(-1), npw, nch)
  xp = jnp.pad(x, ((0, n_pad - n), (0, 0)))
  out = _tc_combine(xp, gl, gr, W_top.T, W_left.T, W_right.T, bm=2 * npw)
  return out[:n]


# speedup vs baseline: 1.9240x; 1.0247x over previous
import functools

import jax
import jax.numpy as jnp
from jax import lax
from jax.experimental import pallas as pl
from jax.experimental.pallas import tpu as pltpu
from jax.experimental.pallas import tpu_sc as plsc

NW = 32
CN = 4
LANES = 16


def _sc_gather_reduce(x, idx_flat, npw, nch):
  n_pad = NW * npw
  f = x.shape[1]
  k = idx_flat.shape[0] // n_pad
  ci = CN * k
  inv = 1.0 / (k - 1)
  mesh = plsc.VectorSubcoreMesh(core_axis_name="c", subcore_axis_name="s")

  @functools.partial(
      pl.kernel,
      out_type=[jax.ShapeDtypeStruct((n_pad, f), jnp.float32),
                jax.ShapeDtypeStruct((n_pad, f), jnp.float32)],
      mesh=mesh,
      scratch_types=[
          pltpu.VMEM((npw * k,), jnp.int32),
          pltpu.VMEM((2, ci, f), jnp.float32),
          pltpu.VMEM((npw, f), jnp.float32),
          pltpu.VMEM((npw, f), jnp.float32),
          pltpu.SemaphoreType.DMA,
          pltpu.SemaphoreType.DMA,
      ],
  )
  def sc_kernel(x_hbm, idx_hbm, gl_hbm, gr_hbm, idxv, rows, glv, grv,
                sem0, sem1):
    sems = (sem0, sem1)
    wid = lax.axis_index("s") * 2 + lax.axis_index("c")
    base = wid * npw

    pltpu.sync_copy(idx_hbm.at[pl.ds(base * k, npw * k)], idxv)

    def gather(c, b):
      return pltpu.make_async_copy(
          x_hbm.at[idxv.at[pl.ds(c * ci, ci)]], rows.at[b], sems[b])

    gather(0, 0).start()
    gather(1, 1).start()

    def do_chunk(c, b):
      gather(c, b).wait()

      nf = f // LANES

      @pl.loop(0, CN)
      def _(i):
        node = c * CN + i
        r0 = i * k
        for g in range(0, nf, 4):
          sls = [pl.ds(fo * LANES, LANES) for fo in range(g, g + 4)]
          ss = [rows[b, r0 + (k - 1), sl] for sl in sls]
          ws = list(ss)
          for j in range(k - 2, 0, -1):
            for fo in range(4):
              v = rows[b, r0 + j, sls[fo]]
              ss[fo] = ss[fo] + v
              ws[fo] = ws[fo] + ss[fo]
          for fo in range(4):
            s = ss[fo] + rows[b, r0, sls[fo]]
            gr = ws[fo] * inv
            glv[node, sls[fo]] = s - gr
            grv[node, sls[fo]] = gr

      nxt = c + 2

      @pl.when(nxt < nch)
      def _():
        gather(nxt, b).start()

    @pl.loop(0, nch, step=2)
    def _(c):
      do_chunk(c, 0)
      do_chunk(c + 1, 1)

    pltpu.sync_copy(glv, gl_hbm.at[pl.ds(base, npw)])
    pltpu.sync_copy(grv, gr_hbm.at[pl.ds(base, npw)])

  return sc_kernel(x, idx_flat)


def _tc_combine(xp, gl, gr, wt_t, wl_t, wr_t, bm):
  n_pad, f = xp.shape

  def body(x_ref, gl_ref, gr_ref, wt_ref, wl_ref, wr_ref, o_ref):
    acc = jnp.dot(x_ref[...], wt_ref[...], preferred_element_type=jnp.float32)
    acc += jnp.dot(gl_ref[...], wl_ref[...], preferred_element_type=jnp.float32)
    acc += jnp.dot(gr_ref[...], wr_ref[...], preferred_element_type=jnp.float32)
    o_ref[...] = acc

  row_spec = pl.BlockSpec((bm, f), lambda i: (i, 0))
  w_spec = pl.BlockSpec((f, f), lambda i: (0, 0))
  return pl.pallas_call(
      body,
      grid=(n_pad // bm,),
      in_specs=[row_spec, row_spec, row_spec, w_spec, w_spec, w_spec],
      out_specs=row_spec,
      out_shape=jax.ShapeDtypeStruct((n_pad, f), jnp.float32),
  )(xp, gl, gr, wt_t, wl_t, wr_t)


def kernel(x, child_idx, W_top, W_left, W_right):
  n, _ = x.shape
  nch = -(-n // (NW * CN))
  nch += nch % 2
  npw = nch * CN
  n_pad = NW * npw
  idx = jnp.pad(child_idx.astype(jnp.int32), ((0, n_pad - n), (0, 0)))
  gl, gr = _sc_gather_reduce(x, idx.reshape(-1), npw, nch)
  xp = jnp.pad(x, ((0, n_pad - n), (0, 0)))
  out = _tc_combine(xp, gl, gr, W_top.T, W_left.T, W_right.T, bm=2 * npw)
  return out[:n]


# speedup vs baseline: 1.9252x; 1.0006x over previous
import functools

import jax
import jax.numpy as jnp
from jax import lax
from jax.experimental import pallas as pl
from jax.experimental.pallas import tpu as pltpu
from jax.experimental.pallas import tpu_sc as plsc

NW = 32
CN = 4
LANES = 16


def _sc_gather_reduce(x, idx_flat, npw, nch):
  n_pad = NW * npw
  f = x.shape[1]
  k = idx_flat.shape[0] // n_pad
  ci = CN * k
  inv = 1.0 / (k - 1)
  mesh = plsc.VectorSubcoreMesh(core_axis_name="c", subcore_axis_name="s")

  @functools.partial(
      pl.kernel,
      out_type=[jax.ShapeDtypeStruct((n_pad, f), jnp.float32),
                jax.ShapeDtypeStruct((n_pad, f), jnp.float32)],
      mesh=mesh,
      scratch_types=[
          pltpu.VMEM((npw * k,), jnp.int32),
          pltpu.VMEM((2, ci, f), jnp.float32),
          pltpu.VMEM((npw, f), jnp.float32),
          pltpu.VMEM((npw, f), jnp.float32),
          pltpu.SemaphoreType.DMA,
          pltpu.SemaphoreType.DMA,
      ],
  )
  def sc_kernel(x_hbm, idx_hbm, gl_hbm, gr_hbm, idxv, rows, glv, grv,
                sem0, sem1):
    sems = (sem0, sem1)
    wid = lax.axis_index("s") * 2 + lax.axis_index("c")
    base = wid * npw

    pltpu.sync_copy(idx_hbm.at[pl.ds(base * k, npw * k)], idxv)

    def gather(c, b):
      return pltpu.make_async_copy(
          x_hbm.at[idxv.at[pl.ds(c * ci, ci)]], rows.at[b], sems[b])

    gather(0, 0).start()
    gather(1, 1).start()

    nf = f // LANES

    @pl.loop(0, nch)
    def _(c):
      b = lax.rem(c, 2)

      @pl.when(b == 0)
      def _():
        gather(c, 0).wait()

      @pl.when(b == 1)
      def _():
        gather(c, 1).wait()

      @pl.loop(0, CN)
      def _(i):
        node = c * CN + i
        r0 = i * k
        for g in range(0, nf, 4):
          sls = [pl.ds(fo * LANES, LANES) for fo in range(g, g + 4)]
          ss = [rows[b, r0 + (k - 1), sl] for sl in sls]
          ws = list(ss)
          for j in range(k - 2, 0, -1):
            for fo in range(4):
              v = rows[b, r0 + j, sls[fo]]
              ss[fo] = ss[fo] + v
              ws[fo] = ws[fo] + ss[fo]
          for fo in range(4):
            s = ss[fo] + rows[b, r0, sls[fo]]
            gr = ws[fo] * inv
            glv[node, sls[fo]] = s - gr
            grv[node, sls[fo]] = gr

      nxt = c + 2

      @pl.when(nxt < nch)
      def _():
        @pl.when(b == 0)
        def _():
          gather(nxt, 0).start()

        @pl.when(b == 1)
        def _():
          gather(nxt, 1).start()

    pltpu.sync_copy(glv, gl_hbm.at[pl.ds(base, npw)])
    pltpu.sync_copy(grv, gr_hbm.at[pl.ds(base, npw)])

  return sc_kernel(x, idx_flat)


def _tc_combine(xp, gl, gr, wt_t, wl_t, wr_t, bm):
  n_pad, f = xp.shape

  def body(x_ref, gl_ref, gr_ref, wt_ref, wl_ref, wr_ref, o_ref):
    acc = jnp.dot(x_ref[...], wt_ref[...], preferred_element_type=jnp.float32)
    acc += jnp.dot(gl_ref[...], wl_ref[...], preferred_element_type=jnp.float32)
    acc += jnp.dot(gr_ref[...], wr_ref[...], preferred_element_type=jnp.float32)
    o_ref[...] = acc

  row_spec = pl.BlockSpec((bm, f), lambda i: (i, 0))
  w_spec = pl.BlockSpec((f, f), lambda i: (0, 0))
  return pl.pallas_call(
      body,
      grid=(n_pad // bm,),
      in_specs=[row_spec, row_spec, row_spec, w_spec, w_spec, w_spec],
      out_specs=row_spec,
      out_shape=jax.ShapeDtypeStruct((n_pad, f), jnp.float32),
  )(xp, gl, gr, wt_t, wl_t, wr_t)


def kernel(x, child_idx, W_top, W_left, W_right):
  n, _ = x.shape
  nch = -(-n // (NW * CN))
  nch += nch % 2
  npw = nch * CN
  n_pad = NW * npw
  idx = jnp.pad(child_idx.astype(jnp.int32), ((0, n_pad - n), (0, 0)))
  gl, gr = _sc_gather_reduce(x, idx.reshape(-1), npw, nch)
  xp = jnp.pad(x, ((0, n_pad - n), (0, 0)))
  out = _tc_combine(xp, gl, gr, W_top.T, W_left.T, W_right.T, bm=2 * npw)
  return out[:n]


# speedup vs baseline: 1.9918x; 1.0346x over previous
import functools

import jax
import jax.numpy as jnp
from jax import lax
from jax.experimental import pallas as pl
from jax.experimental.pallas import tpu as pltpu
from jax.experimental.pallas import tpu_sc as plsc

NW = 32
CN = 4
LANES = 16


def _sc_gather_reduce(x, idx_flat, npw, nch):
  n_pad = NW * npw
  f = x.shape[1]
  k = idx_flat.shape[0] // n_pad
  ci = CN * k
  inv = 1.0 / (k - 1)
  mesh = plsc.VectorSubcoreMesh(core_axis_name="c", subcore_axis_name="s")

  D = 4

  @functools.partial(
      pl.kernel,
      out_type=[jax.ShapeDtypeStruct((n_pad, f), jnp.float32),
                jax.ShapeDtypeStruct((n_pad, f), jnp.float32)],
      mesh=mesh,
      scratch_types=[
          pltpu.VMEM((npw * k,), jnp.int32),
          pltpu.VMEM((D, ci, f), jnp.float32),
          pltpu.VMEM((2, CN, f), jnp.float32),
          pltpu.VMEM((2, CN, f), jnp.float32),
          [pltpu.SemaphoreType.DMA] * D,
          [pltpu.SemaphoreType.DMA] * 2,
          [pltpu.SemaphoreType.DMA] * 2,
      ],
  )
  def sc_kernel(x_hbm, idx_hbm, gl_hbm, gr_hbm, idxv, rows, glo, gro,
                gsems, lsems, rsems):
    wid = lax.axis_index("s") * 2 + lax.axis_index("c")
    base = wid * npw

    pltpu.sync_copy(idx_hbm.at[pl.ds(base * k, npw * k)], idxv)

    def gather(c, b):
      return pltpu.make_async_copy(
          x_hbm.at[idxv.at[pl.ds(c * ci, ci)]], rows.at[b], gsems[b])

    def store(c, ob):
      dst = pl.ds(base + c * CN, CN)
      return (pltpu.make_async_copy(glo.at[ob], gl_hbm.at[dst], lsems[ob]),
              pltpu.make_async_copy(gro.at[ob], gr_hbm.at[dst], rsems[ob]))

    for b in range(D):
      gather(b, b).start()

    nf = f // LANES

    @pl.loop(0, nch)
    def _(c):
      b = lax.rem(c, D)
      ob = lax.rem(c, 2)
      for bb in range(D):
        @pl.when(b == bb)
        def _():
          gather(c, bb).wait()

      @pl.when(c >= 2)
      def _():
        for oo in range(2):
          @pl.when(ob == oo)
          def _():
            s_l, s_r = store(c - 2, oo)
            s_l.wait()
            s_r.wait()

      @pl.loop(0, CN)
      def _(i):
        r0 = i * k
        for g in range(0, nf, 4):
          sls = [pl.ds(fo * LANES, LANES) for fo in range(g, g + 4)]
          ss = [rows[b, r0 + (k - 1), sl] for sl in sls]
          ws = list(ss)
          for j in range(k - 2, 0, -1):
            for fo in range(4):
              v = rows[b, r0 + j, sls[fo]]
              ss[fo] = ss[fo] + v
              ws[fo] = ws[fo] + ss[fo]
          for fo in range(4):
            s = ss[fo] + rows[b, r0, sls[fo]]
            gr = ws[fo] * inv
            glo[ob, i, sls[fo]] = s - gr
            gro[ob, i, sls[fo]] = gr

      for oo in range(2):
        @pl.when(ob == oo)
        def _():
          s_l, s_r = store(c, oo)
          s_l.start()
          s_r.start()

      nxt = c + D

      @pl.when(nxt < nch)
      def _():
        for bb in range(D):
          @pl.when(b == bb)
          def _():
            gather(nxt, bb).start()

    for cc in (nch - 2, nch - 1):
      s_l, s_r = store(cc, cc % 2)
      s_l.wait()
      s_r.wait()

  return sc_kernel(x, idx_flat)


def _tc_combine(xp, gl, gr, wt_t, wl_t, wr_t, bm):
  n_pad, f = xp.shape

  def body(x_ref, gl_ref, gr_ref, wt_ref, wl_ref, wr_ref, o_ref):
    acc = jnp.dot(x_ref[...], wt_ref[...], preferred_element_type=jnp.float32)
    acc += jnp.dot(gl_ref[...], wl_ref[...], preferred_element_type=jnp.float32)
    acc += jnp.dot(gr_ref[...], wr_ref[...], preferred_element_type=jnp.float32)
    o_ref[...] = acc

  row_spec = pl.BlockSpec((bm, f), lambda i: (i, 0))
  w_spec = pl.BlockSpec((f, f), lambda i: (0, 0))
  return pl.pallas_call(
      body,
      grid=(n_pad // bm,),
      in_specs=[row_spec, row_spec, row_spec, w_spec, w_spec, w_spec],
      out_specs=row_spec,
      out_shape=jax.ShapeDtypeStruct((n_pad, f), jnp.float32),
  )(xp, gl, gr, wt_t, wl_t, wr_t)


def kernel(x, child_idx, W_top, W_left, W_right):
  n, _ = x.shape
  nch = -(-n // (NW * CN))
  nch += nch % 2
  npw = nch * CN
  n_pad = NW * npw
  idx = jnp.pad(child_idx.astype(jnp.int32), ((0, n_pad - n), (0, 0)))
  gl, gr = _sc_gather_reduce(x, idx.reshape(-1), npw, nch)
  xp = jnp.pad(x, ((0, n_pad - n), (0, 0)))
  out = _tc_combine(xp, gl, gr, W_top.T, W_left.T, W_right.T, bm=2 * npw)
  return out[:n]


# speedup vs baseline: 6.5651x; 3.2960x over previous
import functools

import jax
import jax.numpy as jnp
from jax import lax
from jax.experimental import pallas as pl
from jax.experimental.pallas import tpu as pltpu
from jax.experimental.pallas import tpu_sc as plsc

NW = 32
TPS = 16
CN = 4
LANES = 16
D = 2
OD = 4


def _sc_gather_reduce(x, idx_flat, npw, nch):
  n_pad = NW * npw
  nx, f = x.shape
  k = idx_flat.shape[0] // n_pad
  ci = CN * k
  inv = 1.0 / (k - 1)
  stripe = nx // TPS
  mesh = plsc.VectorSubcoreMesh(core_axis_name="c", subcore_axis_name="s")

  @functools.partial(
      pl.kernel,
      out_type=[jax.ShapeDtypeStruct((n_pad, f), jnp.float32),
                jax.ShapeDtypeStruct((n_pad, f), jnp.float32)],
      mesh=mesh,
      scratch_types=[
          pltpu.VMEM_SHARED((nx, f), jnp.float32),
          pltpu.VMEM((npw * k,), jnp.int32),
          pltpu.VMEM((D, ci, f), jnp.float32),
          pltpu.VMEM((OD, CN, f), jnp.float32),
          pltpu.VMEM((OD, CN, f), jnp.float32),
          [pltpu.SemaphoreType.DMA] * D,
          [pltpu.SemaphoreType.DMA] * OD,
          [pltpu.SemaphoreType.DMA] * OD,
      ],
  )
  def sc_kernel(x_hbm, idx_hbm, gl_hbm, gr_hbm, xs, idxv, rows, glo, gro,
                gsems, lsems, rsems):
    wid = lax.axis_index("s") * 2 + lax.axis_index("c")
    sid = lax.axis_index("s")
    base = wid * npw

    pltpu.sync_copy(idx_hbm.at[pl.ds(base * k, npw * k)], idxv)

    nstage = stripe // ci

    def stage_in(p, b):
      row0 = sid * stripe + p * ci
      return pltpu.make_async_copy(x_hbm.at[pl.ds(row0, ci)], rows.at[b],
                                   gsems[b])

    def stage_out(p, b):
      row0 = sid * stripe + p * ci
      return pltpu.make_async_copy(rows.at[b], xs.at[pl.ds(row0, ci)],
                                   lsems[b])

    stage_in(0, 0).start()
    for p in range(nstage):
      b = p % 2
      stage_in(p, b).wait()
      stage_out(p, b).start()
      if p + 1 < nstage:
        if p >= 1:
          stage_out(p - 1, 1 - b).wait()
        stage_in(p + 1, 1 - b).start()
    stage_out(nstage - 1, (nstage - 1) % 2).wait()

    plsc.subcore_barrier()

    def gather(c, b):
      return pltpu.make_async_copy(
          xs.at[idxv.at[pl.ds(c * ci, ci)]], rows.at[b], gsems[b])

    def store(c, ob):
      dst = pl.ds(base + c * CN, CN)
      return (pltpu.make_async_copy(glo.at[ob], gl_hbm.at[dst], lsems[ob]),
              pltpu.make_async_copy(gro.at[ob], gr_hbm.at[dst], rsems[ob]))

    for b in range(D):
      gather(b, b).start()

    nf = f // LANES

    @pl.loop(0, nch)
    def _(c):
      b = lax.rem(c, D)
      ob = lax.rem(c, OD)
      for bb in range(D):
        @pl.when(b == bb)
        def _():
          gather(c, bb).wait()

      @pl.when(c >= OD)
      def _():
        for oo in range(OD):
          @pl.when(ob == oo)
          def _():
            s_l, s_r = store(c - OD, oo)
            s_l.wait()
            s_r.wait()

      @pl.loop(0, CN)
      def _(i):
        r0 = i * k
        for g in range(0, nf, 4):
          sls = [pl.ds(fo * LANES, LANES) for fo in range(g, g + 4)]
          ss = [rows[b, r0 + (k - 1), sl] for sl in sls]
          ws = list(ss)
          for j in range(k - 2, 0, -1):
            for fo in range(4):
              v = rows[b, r0 + j, sls[fo]]
              ss[fo] = ss[fo] + v
              ws[fo] = ws[fo] + ss[fo]
          for fo in range(4):
            s = ss[fo] + rows[b, r0, sls[fo]]
            gr = ws[fo] * inv
            glo[ob, i, sls[fo]] = s - gr
            gro[ob, i, sls[fo]] = gr

      for oo in range(OD):
        @pl.when(ob == oo)
        def _():
          s_l, s_r = store(c, oo)
          s_l.start()
          s_r.start()

      nxt = c + D

      @pl.when(nxt < nch)
      def _():
        for bb in range(D):
          @pl.when(b == bb)
          def _():
            gather(nxt, bb).start()

    for cc in range(max(nch - OD, 0), nch):
      s_l, s_r = store(cc, cc % OD)
      s_l.wait()
      s_r.wait()

  return sc_kernel(x, idx_flat)


def _tc_combine(xp, gl, gr, wt_t, wl_t, wr_t, bm):
  n_pad, f = xp.shape

  def body(x_ref, gl_ref, gr_ref, wt_ref, wl_ref, wr_ref, o_ref):
    acc = jnp.dot(x_ref[...], wt_ref[...], preferred_element_type=jnp.float32)
    acc += jnp.dot(gl_ref[...], wl_ref[...], preferred_element_type=jnp.float32)
    acc += jnp.dot(gr_ref[...], wr_ref[...], preferred_element_type=jnp.float32)
    o_ref[...] = acc

  row_spec = pl.BlockSpec((bm, f), lambda i: (i, 0))
  w_spec = pl.BlockSpec((f, f), lambda i: (0, 0))
  return pl.pallas_call(
      body,
      grid=(n_pad // bm,),
      in_specs=[row_spec, row_spec, row_spec, w_spec, w_spec, w_spec],
      out_specs=row_spec,
      out_shape=jax.ShapeDtypeStruct((n_pad, f), jnp.float32),
  )(xp, gl, gr, wt_t, wl_t, wr_t)


def kernel(x, child_idx, W_top, W_left, W_right):
  n, _ = x.shape
  nch = -(-n // (NW * CN))
  nch = -(-nch // OD) * OD
  npw = nch * CN
  n_pad = NW * npw
  idx = jnp.pad(child_idx.astype(jnp.int32), ((0, n_pad - n), (0, 0)))
  xp = jnp.pad(x, ((0, n_pad - n), (0, 0)))
  gl, gr = _sc_gather_reduce(xp, idx.reshape(-1), npw, nch)
  out = _tc_combine(xp, gl, gr, W_top.T, W_left.T, W_right.T, bm=2 * npw)
  return out[:n]


# speedup vs baseline: 7.3730x; 1.1231x over previous
import functools

import jax
import jax.numpy as jnp
from jax import lax
from jax.experimental import pallas as pl
from jax.experimental.pallas import tpu as pltpu
from jax.experimental.pallas import tpu_sc as plsc

NW = 32
TPS = 16
CN = 4
LANES = 16
D = 2
OD = 4


def _sc_gather_reduce(x, idx_flat, npw, nch, nx):
  n_pad = NW * npw
  n, f = x.shape
  k = idx_flat.shape[0] // n_pad
  ci = CN * k
  inv = 1.0 / (k - 1)
  nfp = n // ci
  full_rounds = nfp // TPS
  rem_p = nfp % TPS
  tail = n - nfp * ci
  mesh = plsc.VectorSubcoreMesh(core_axis_name="c", subcore_axis_name="s")

  @functools.partial(
      pl.kernel,
      out_type=[jax.ShapeDtypeStruct((n_pad, f), jnp.float32),
                jax.ShapeDtypeStruct((n_pad, f), jnp.float32)],
      mesh=mesh,
      scratch_types=[
          pltpu.VMEM_SHARED((nx, f), jnp.float32),
          pltpu.VMEM((npw * k,), jnp.int32),
          pltpu.VMEM((D, ci, f), jnp.float32),
          pltpu.VMEM((OD, CN, f), jnp.float32),
          pltpu.VMEM((OD, CN, f), jnp.float32),
          [pltpu.SemaphoreType.DMA] * D,
          [pltpu.SemaphoreType.DMA] * OD,
          [pltpu.SemaphoreType.DMA] * OD,
      ],
  )
  def sc_kernel(x_hbm, idx_hbm, gl_hbm, gr_hbm, xs, idxv, rows, glo, gro,
                gsems, lsems, rsems):
    wid = lax.axis_index("s") * 2 + lax.axis_index("c")
    sid = lax.axis_index("s")
    base = wid * npw

    pltpu.sync_copy(idx_hbm.at[pl.ds(base * k, npw * k)], idxv)

    def stage_in(p, b):
      row0 = (p * TPS + sid) * ci
      return pltpu.make_async_copy(x_hbm.at[pl.ds(row0, ci)], rows.at[b],
                                   gsems[b])

    def stage_out(p, b):
      row0 = (p * TPS + sid) * ci
      return pltpu.make_async_copy(rows.at[b], xs.at[pl.ds(row0, ci)],
                                   lsems[b])

    if full_rounds:
      stage_in(0, 0).start()
      for p in range(full_rounds):
        b = p % 2
        stage_in(p, b).wait()
        stage_out(p, b).start()
        if p + 1 < full_rounds:
          if p >= 1:
            stage_out(p - 1, 1 - b).wait()
          stage_in(p + 1, 1 - b).start()
      if full_rounds >= 2:
        stage_out(full_rounds - 2, full_rounds % 2).wait()
      stage_out(full_rounds - 1, (full_rounds - 1) % 2).wait()

    if rem_p:
      @pl.when(sid < rem_p)
      def _():
        row0 = (full_rounds * TPS + sid) * ci
        pltpu.sync_copy(x_hbm.at[pl.ds(row0, ci)], rows.at[0])
        pltpu.sync_copy(rows.at[0], xs.at[pl.ds(row0, ci)])

    if tail:
      @pl.when(sid == TPS - 1)
      def _():
        row0 = nfp * ci
        pltpu.sync_copy(x_hbm.at[pl.ds(row0, tail)],
                        rows.at[0, pl.ds(0, tail)])
        pltpu.sync_copy(rows.at[0, pl.ds(0, tail)],
                        xs.at[pl.ds(row0, tail)])

    plsc.subcore_barrier()

    def gather(c, b):
      return pltpu.make_async_copy(
          xs.at[idxv.at[pl.ds(c * ci, ci)]], rows.at[b], gsems[b])

    def store(c, ob):
      dst = pl.ds(base + c * CN, CN)
      return (pltpu.make_async_copy(glo.at[ob], gl_hbm.at[dst], lsems[ob]),
              pltpu.make_async_copy(gro.at[ob], gr_hbm.at[dst], rsems[ob]))

    for b in range(D):
      gather(b, b).start()

    nf = f // LANES

    @pl.loop(0, nch)
    def _(c):
      b = lax.rem(c, D)
      ob = lax.rem(c, OD)
      for bb in range(D):
        @pl.when(b == bb)
        def _():
          gather(c, bb).wait()

      @pl.when(c >= OD)
      def _():
        for oo in range(OD):
          @pl.when(ob == oo)
          def _():
            s_l, s_r = store(c - OD, oo)
            s_l.wait()
            s_r.wait()

      @pl.loop(0, CN)
      def _(i):
        r0 = i * k
        for g in range(0, nf, 4):
          sls = [pl.ds(fo * LANES, LANES) for fo in range(g, g + 4)]
          ss = [rows[b, r0 + (k - 1), sl] for sl in sls]
          ws = list(ss)
          for j in range(k - 2, 0, -1):
            for fo in range(4):
              v = rows[b, r0 + j, sls[fo]]
              ss[fo] = ss[fo] + v
              ws[fo] = ws[fo] + ss[fo]
          for fo in range(4):
            s = ss[fo] + rows[b, r0, sls[fo]]
            gr = ws[fo] * inv
            glo[ob, i, sls[fo]] = s - gr
            gro[ob, i, sls[fo]] = gr

      for oo in range(OD):
        @pl.when(ob == oo)
        def _():
          s_l, s_r = store(c, oo)
          s_l.start()
          s_r.start()

      nxt = c + D

      @pl.when(nxt < nch)
      def _():
        for bb in range(D):
          @pl.when(b == bb)
          def _():
            gather(nxt, bb).start()

    for cc in range(max(nch - OD, 0), nch):
      s_l, s_r = store(cc, cc % OD)
      s_l.wait()
      s_r.wait()

  return sc_kernel(x, idx_flat)


def _tc_combine(x, gl, gr, wt_t, wl_t, wr_t, bm):
  n, f = x.shape

  def body(x_ref, gl_ref, gr_ref, wt_ref, wl_ref, wr_ref, o_ref):
    acc = jnp.dot(x_ref[...], wt_ref[...], preferred_element_type=jnp.float32)
    acc += jnp.dot(gl_ref[...], wl_ref[...], preferred_element_type=jnp.float32)
    acc += jnp.dot(gr_ref[...], wr_ref[...], preferred_element_type=jnp.float32)
    o_ref[...] = acc

  row_spec = pl.BlockSpec((bm, f), lambda i: (i, 0))
  w_spec = pl.BlockSpec((f, f), lambda i: (0, 0))
  return pl.pallas_call(
      body,
      grid=(-(-n // bm),),
      in_specs=[row_spec, row_spec, row_spec, w_spec, w_spec, w_spec],
      out_specs=row_spec,
      out_shape=jax.ShapeDtypeStruct((n, f), jnp.float32),
  )(x, gl, gr, wt_t, wl_t, wr_t)


def kernel(x, child_idx, W_top, W_left, W_right):
  n, _ = x.shape
  nch = -(-n // (NW * CN))
  nch = -(-nch // OD) * OD
  npw = nch * CN
  n_pad = NW * npw
  idx = jnp.pad(child_idx.astype(jnp.int32), ((0, n_pad - n), (0, 0)))
  if n % 8 == 0:
    xs_src = x
  else:
    xs_src = jnp.pad(x, ((0, 8 - n % 8), (0, 0)))
  gl, gr = _sc_gather_reduce(xs_src, idx.reshape(-1), npw, nch, n_pad)
  bm = 2000 if n % 2000 == 0 else 640
  out = _tc_combine(x, gl, gr, W_top.T, W_left.T, W_right.T, bm=bm)
  return out


# speedup vs baseline: 7.6237x; 1.0340x over previous
import functools

import jax
import jax.numpy as jnp
from jax import lax
from jax.experimental import pallas as pl
from jax.experimental.pallas import tpu as pltpu
from jax.experimental.pallas import tpu_sc as plsc

NW = 32
TPS = 16
CN = 4
LANES = 16
D = 2
OD = 4


def _sc_gather_reduce(x, idx_flat, npw, nch, nx, k):
  n_pad = NW * npw
  n, f = x.shape
  vlast = idx_flat.shape[0] - (NW - 1) * npw * k
  ci = CN * k
  inv = 1.0 / (k - 1)
  nfp = n // ci
  full_rounds = nfp // TPS
  rem_p = nfp % TPS
  tail = n - nfp * ci
  mesh = plsc.VectorSubcoreMesh(core_axis_name="c", subcore_axis_name="s")

  @functools.partial(
      pl.kernel,
      out_type=[jax.ShapeDtypeStruct((n_pad, f), jnp.float32),
                jax.ShapeDtypeStruct((n_pad, f), jnp.float32)],
      mesh=mesh,
      scratch_types=[
          pltpu.VMEM_SHARED((nx, f), jnp.float32),
          pltpu.VMEM((npw * k,), jnp.int32),
          pltpu.VMEM((D, ci, f), jnp.float32),
          pltpu.VMEM((OD, CN, f), jnp.float32),
          pltpu.VMEM((OD, CN, f), jnp.float32),
          [pltpu.SemaphoreType.DMA] * D,
          [pltpu.SemaphoreType.DMA] * OD,
          [pltpu.SemaphoreType.DMA] * OD,
      ],
  )
  def sc_kernel(x_hbm, idx_hbm, gl_hbm, gr_hbm, xs, idxv, rows, glo, gro,
                gsems, lsems, rsems):
    wid = lax.axis_index("s") * 2 + lax.axis_index("c")
    sid = lax.axis_index("s")
    base = wid * npw

    if vlast == npw * k:
      pltpu.sync_copy(idx_hbm.at[pl.ds(base * k, npw * k)], idxv)
    else:
      @pl.when(wid < NW - 1)
      def _():
        pltpu.sync_copy(idx_hbm.at[pl.ds(base * k, npw * k)], idxv)

      @pl.when(wid == NW - 1)
      def _():
        pltpu.sync_copy(idx_hbm.at[pl.ds(base * k, vlast)],
                        idxv.at[pl.ds(0, vlast)])
        zero = jnp.zeros((LANES,), jnp.int32)

        @pl.loop(0, (npw * k - vlast) // LANES)
        def _(z):
          idxv[pl.ds(vlast + z * LANES, LANES)] = zero

    def stage_in(p, b):
      row0 = (p * TPS + sid) * ci
      return pltpu.make_async_copy(x_hbm.at[pl.ds(row0, ci)], rows.at[b],
                                   gsems[b])

    def stage_out(p, b):
      row0 = (p * TPS + sid) * ci
      return pltpu.make_async_copy(rows.at[b], xs.at[pl.ds(row0, ci)],
                                   lsems[b])

    if full_rounds:
      stage_in(0, 0).start()
      for p in range(full_rounds):
        b = p % 2
        stage_in(p, b).wait()
        stage_out(p, b).start()
        if p + 1 < full_rounds:
          if p >= 1:
            stage_out(p - 1, 1 - b).wait()
          stage_in(p + 1, 1 - b).start()
      if full_rounds >= 2:
        stage_out(full_rounds - 2, full_rounds % 2).wait()
      stage_out(full_rounds - 1, (full_rounds - 1) % 2).wait()

    if rem_p:
      @pl.when(sid < rem_p)
      def _():
        row0 = (full_rounds * TPS + sid) * ci
        pltpu.sync_copy(x_hbm.at[pl.ds(row0, ci)], rows.at[0])
        pltpu.sync_copy(rows.at[0], xs.at[pl.ds(row0, ci)])

    if tail:
      @pl.when(sid == TPS - 1)
      def _():
        row0 = nfp * ci
        pltpu.sync_copy(x_hbm.at[pl.ds(row0, tail)],
                        rows.at[0, pl.ds(0, tail)])
        pltpu.sync_copy(rows.at[0, pl.ds(0, tail)],
                        xs.at[pl.ds(row0, tail)])

    plsc.subcore_barrier()

    def gather(c, b):
      return pltpu.make_async_copy(
          xs.at[idxv.at[pl.ds(c * ci, ci)]], rows.at[b], gsems[b])

    def store(c, ob):
      dst = pl.ds(base + c * CN, CN)
      return (pltpu.make_async_copy(glo.at[ob], gl_hbm.at[dst], lsems[ob]),
              pltpu.make_async_copy(gro.at[ob], gr_hbm.at[dst], rsems[ob]))

    for b in range(D):
      gather(b, b).start()

    nf = f // LANES

    @pl.loop(0, nch)
    def _(c):
      b = lax.rem(c, D)
      ob = lax.rem(c, OD)
      for bb in range(D):
        @pl.when(b == bb)
        def _():
          gather(c, bb).wait()

      @pl.when(c >= OD)
      def _():
        for oo in range(OD):
          @pl.when(ob == oo)
          def _():
            s_l, s_r = store(c - OD, oo)
            s_l.wait()
            s_r.wait()

      @pl.loop(0, CN)
      def _(i):
        r0 = i * k
        for g in range(0, nf, 4):
          sls = [pl.ds(fo * LANES, LANES) for fo in range(g, g + 4)]
          ss = [rows[b, r0 + (k - 1), sl] for sl in sls]
          ws = list(ss)
          for j in range(k - 2, 0, -1):
            for fo in range(4):
              v = rows[b, r0 + j, sls[fo]]
              ss[fo] = ss[fo] + v
              ws[fo] = ws[fo] + ss[fo]
          for fo in range(4):
            s = ss[fo] + rows[b, r0, sls[fo]]
            gr = ws[fo] * inv
            glo[ob, i, sls[fo]] = s - gr
            gro[ob, i, sls[fo]] = gr

      for oo in range(OD):
        @pl.when(ob == oo)
        def _():
          s_l, s_r = store(c, oo)
          s_l.start()
          s_r.start()

      nxt = c + D

      @pl.when(nxt < nch)
      def _():
        for bb in range(D):
          @pl.when(b == bb)
          def _():
            gather(nxt, bb).start()

    for cc in range(max(nch - OD, 0), nch):
      s_l, s_r = store(cc, cc % OD)
      s_l.wait()
      s_r.wait()

  return sc_kernel(x, idx_flat)


def _tc_combine(x, gl, gr, wt_t, wl_t, wr_t, bm):
  n, f = x.shape

  def body(x_ref, gl_ref, gr_ref, wt_ref, wl_ref, wr_ref, o_ref):
    acc = jnp.dot(x_ref[...], wt_ref[...], preferred_element_type=jnp.float32)
    acc += jnp.dot(gl_ref[...], wl_ref[...], preferred_element_type=jnp.float32)
    acc += jnp.dot(gr_ref[...], wr_ref[...], preferred_element_type=jnp.float32)
    o_ref[...] = acc

  row_spec = pl.BlockSpec((bm, f), lambda i: (i, 0))
  w_spec = pl.BlockSpec((f, f), lambda i: (0, 0))
  return pl.pallas_call(
      body,
      grid=(-(-n // bm),),
      in_specs=[row_spec, row_spec, row_spec, w_spec, w_spec, w_spec],
      out_specs=row_spec,
      out_shape=jax.ShapeDtypeStruct((n, f), jnp.float32),
  )(x, gl, gr, wt_t, wl_t, wr_t)


def kernel(x, child_idx, W_top, W_left, W_right):
  n, _ = x.shape
  nch = -(-n // (NW * CN))
  nch = -(-nch // OD) * OD
  npw = nch * CN
  n_pad = NW * npw
  k = child_idx.shape[1]
  r_last = n - (NW - 1) * npw
  if 0 < r_last <= npw and (r_last * k) % LANES == 0:
    idx_flat = child_idx.astype(jnp.int32).reshape(-1)
  else:
    idx_flat = jnp.pad(child_idx.astype(jnp.int32),
                       ((0, n_pad - n), (0, 0))).reshape(-1)
  if n % 8 == 0:
    xs_src = x
  else:
    xs_src = jnp.pad(x, ((0, 8 - n % 8), (0, 0)))
  gl, gr = _sc_gather_reduce(xs_src, idx_flat, npw, nch, n_pad, k)
  bm = 2000 if n % 2000 == 0 else 640
  out = _tc_combine(x, gl, gr, W_top.T, W_left.T, W_right.T, bm=bm)
  return out
